# baseline (speedup 1.0000x reference)
import functools

import jax
import jax.numpy as jnp
from jax import lax
from jax.experimental import pallas as pl
from jax.experimental.pallas import tpu as pltpu
from jax.experimental.pallas import tpu_sc as plsc

N = 10000
E = 320000
D = 128
EPS = 1e-5

NC = 2
NS = 16
NW = NC * NS
CW = 125
E2 = E // CW
ROWS_W = E2 // NW
N_PAD = 10240
CNT_W = N_PAD // NS
AGG_W = N_PAD // NS
ZROWS = 64
IDXB = 16

_mesh = plsc.VectorSubcoreMesh(core_axis_name="c", subcore_axis_name="s")


@functools.partial(
    pl.kernel,
    out_type=[
        jax.ShapeDtypeStruct((NC * N_PAD,), jnp.float32),
        jax.ShapeDtypeStruct((NC * N_PAD,), jnp.float32),
    ],
    mesh=_mesh,
    scratch_types=[
        pltpu.VMEM((ROWS_W, CW), jnp.int32),
        pltpu.VMEM((128,), jnp.float32),
        pltpu.VMEM((CNT_W,), jnp.float32),
        pltpu.VMEM_SHARED((N_PAD,), jnp.float32),
        pltpu.VMEM_SHARED((N_PAD,), jnp.float32),
    ],
)
def _degree_kernel(src_hbm, dst_hbm, out_src, out_dst,
                   idx_v, ones_v, buf_v, cs_sh, cd_sh):
    c = lax.axis_index("c")
    s = lax.axis_index("s")
    w = c * NS + s

    z16 = jnp.zeros((16,), jnp.float32)
    for i in range(CNT_W // 16):
        buf_v[pl.ds(i * 16, 16)] = z16
    o16 = jnp.ones((16,), jnp.float32)
    for i in range(128 // 16):
        ones_v[pl.ds(i * 16, 16)] = o16

    cnt0 = s * CNT_W
    pltpu.sync_copy(buf_v, cs_sh.at[pl.ds(cnt0, CNT_W)])
    pltpu.sync_copy(buf_v, cd_sh.at[pl.ds(cnt0, CNT_W)])
    plsc.subcore_barrier()

    base = w * ROWS_W
    ones_cw = ones_v.at[pl.ds(0, CW)]

    pltpu.sync_copy(src_hbm.at[pl.ds(base, ROWS_W)], idx_v)

    def body_s(j, carry):
        pltpu.sync_copy(ones_cw, cs_sh.at[idx_v.at[j]], add=True)
        return carry

    lax.fori_loop(0, ROWS_W, body_s, 0)

    pltpu.sync_copy(dst_hbm.at[pl.ds(base, ROWS_W)], idx_v)

    def body_d(j, carry):
        pltpu.sync_copy(ones_cw, cd_sh.at[idx_v.at[j]], add=True)
        return carry

    lax.fori_loop(0, ROWS_W, body_d, 0)

    plsc.subcore_barrier()
    out0 = c * N_PAD + cnt0
    pltpu.sync_copy(cs_sh.at[pl.ds(cnt0, CNT_W)], buf_v)
    pltpu.sync_copy(buf_v, out_src.at[pl.ds(out0, CNT_W)])
    pltpu.sync_copy(cd_sh.at[pl.ds(cnt0, CNT_W)], buf_v)
    pltpu.sync_copy(buf_v, out_dst.at[pl.ds(out0, CNT_W)])


@functools.partial(
    pl.kernel,
    out_type=jax.ShapeDtypeStruct((NC, N_PAD, D), jnp.float32),
    mesh=_mesh,
    scratch_types=[
        pltpu.VMEM((2, IDXB, CW), jnp.int32),
        pltpu.VMEM((2, IDXB, CW), jnp.int32),
        pltpu.VMEM((CW, D), jnp.float32),
        pltpu.VMEM((CW, D), jnp.float32),
        pltpu.VMEM_SHARED((N_PAD, D), jnp.float32),
        pltpu.SemaphoreType.DMA,
        pltpu.SemaphoreType.DMA,
        pltpu.SemaphoreType.DMA,
        pltpu.SemaphoreType.DMA,
        pltpu.SemaphoreType.DMA,
        pltpu.SemaphoreType.DMA,
    ],
)
def _agg_kernel(h_hbm, src_hbm, dst_hbm, zeros_hbm, out_hbm,
                sidx_b, didx_b, ra, rb, agg_sh, s0, s1, si, di, t0, t1):
    c = lax.axis_index("c")
    s = lax.axis_index("s")
    w = c * NS + s
    row0 = s * AGG_W
    base = w * ROWS_W
    bufs = (ra, rb)
    sems = (s0, s1)

    zra = ra.at[pl.ds(0, ZROWS)]
    pltpu.sync_copy(zeros_hbm, zra)
    for i in range(AGG_W // ZROWS):
        pltpu.sync_copy(zra, agg_sh.at[pl.ds(row0 + i * ZROWS, ZROWS)])
    plsc.subcore_barrier()

    def idx_copy(b):
        return (
            pltpu.make_async_copy(
                src_hbm.at[pl.ds(base + b * IDXB, IDXB)], sidx_b.at[b % 2], si),
            pltpu.make_async_copy(
                dst_hbm.at[pl.ds(base + b * IDXB, IDXB)], didx_b.at[b % 2], di),
        )

    def gather_copy(j):
        b, pos = j // IDXB, j % IDXB
        return pltpu.make_async_copy(
            h_hbm.at[sidx_b.at[b % 2, pos]], bufs[j % 2], sems[j % 2])

    def scatter_copy(j):
        b, pos = j // IDXB, j % IDXB
        return pltpu.make_async_copy(
            bufs[j % 2], agg_sh.at[didx_b.at[b % 2, pos]], (t0, t1)[j % 2])

    for cp in idx_copy(0):
        cp.start()
    for cp in idx_copy(0):
        cp.wait()
    for cp in idx_copy(1):
        cp.start()
    gather_copy(0).start()
    for j in range(ROWS_W):
        gather_copy(j).wait()
        scatter_copy(j).start(add=True)
        if j > 0:
            scatter_copy(j - 1).wait()
        if j % IDXB == 0 and 0 < j and j // IDXB + 1 < ROWS_W // IDXB:
            for cp in idx_copy(j // IDXB + 1):
                cp.start()
        if j + 1 < ROWS_W:
            if (j + 1) % IDXB == 0:
                for cp in idx_copy((j + 1) // IDXB):
                    cp.wait()
            gather_copy(j + 1).start()
    scatter_copy(ROWS_W - 1).wait()

    plsc.subcore_barrier()

    def out_read(i):
        return pltpu.make_async_copy(
            agg_sh.at[pl.ds(row0 + i * ZROWS, ZROWS)],
            bufs[i % 2].at[pl.ds(0, ZROWS)], sems[i % 2])

    out_read(0).start()
    for i in range(AGG_W // ZROWS):
        if i + 1 < AGG_W // ZROWS:
            out_read(i + 1).start()
        out_read(i).wait()
        pltpu.sync_copy(bufs[i % 2].at[pl.ds(0, ZROWS)],
                        out_hbm.at[c, pl.ds(row0 + i * ZROWS, ZROWS)])


BN = 1000


def _gelu(v):
    return 0.5 * v * (1.0 + lax.erf(v * 0.7071067811865476))


def _norm_scale_body(cs_ref, cd_ref, x_ref, h_ref, ns_ref, nd_ref):
    deg_s = jnp.maximum(cs_ref[...].sum(axis=1, keepdims=True), 1.0)
    deg_d = jnp.maximum(cd_ref[...].sum(axis=1, keepdims=True), 1.0)
    ns = lax.rsqrt(deg_s)
    nd = lax.rsqrt(deg_d)
    ns_ref[...] = ns
    nd_ref[...] = nd
    h_ref[...] = x_ref[...] * ns


_norm_scale = pl.pallas_call(
    _norm_scale_body,
    grid=(N // BN,),
    in_specs=[
        pl.BlockSpec((BN, 2), lambda i: (i, 0)),
        pl.BlockSpec((BN, 2), lambda i: (i, 0)),
        pl.BlockSpec((BN, D), lambda i: (i, 0)),
    ],
    out_specs=[
        pl.BlockSpec((BN, D), lambda i: (i, 0)),
        pl.BlockSpec((BN, 1), lambda i: (i, 0)),
        pl.BlockSpec((BN, 1), lambda i: (i, 0)),
    ],
    out_shape=[
        jax.ShapeDtypeStruct((N, D), jnp.float32),
        jax.ShapeDtypeStruct((N, 1), jnp.float32),
        jax.ShapeDtypeStruct((N, 1), jnp.float32),
    ],
)


def _layer_body(p0_ref, p1_ref, nd_ref, ns_ref, w_ref, b_ref, out_ref):
    agg = (p0_ref[0] + p1_ref[0]) * nd_ref[...]
    h = jnp.dot(agg, w_ref[...], preferred_element_type=jnp.float32) + b_ref[...]
    out_ref[...] = _gelu(h) * ns_ref[...]


_layer_mm = pl.pallas_call(
    _layer_body,
    grid=(N // BN,),
    in_specs=[
        pl.BlockSpec((1, BN, D), lambda i: (0, i, 0)),
        pl.BlockSpec((1, BN, D), lambda i: (1, i, 0)),
        pl.BlockSpec((BN, 1), lambda i: (i, 0)),
        pl.BlockSpec((BN, 1), lambda i: (i, 0)),
        pl.BlockSpec((D, D), lambda i: (0, 0)),
        pl.BlockSpec((1, D), lambda i: (0, 0)),
    ],
    out_specs=pl.BlockSpec((BN, D), lambda i: (i, 0)),
    out_shape=jax.ShapeDtypeStruct((N, D), jnp.float32),
)


def _final_body(p0_ref, p1_ref, nd_ref, w_ref, b_ref, g_ref, be_ref, out_ref):
    agg = (p0_ref[0] + p1_ref[0]) * nd_ref[...]
    h = jnp.dot(agg, w_ref[...], preferred_element_type=jnp.float32) + b_ref[...]
    mu = jnp.mean(h, axis=-1, keepdims=True)
    cent = h - mu
    var = jnp.mean(cent * cent, axis=-1, keepdims=True)
    out_ref[...] = cent * lax.rsqrt(var + EPS) * g_ref[...] + be_ref[...]


_final_mm = pl.pallas_call(
    _final_body,
    grid=(N // BN,),
    in_specs=[
        pl.BlockSpec((1, BN, D), lambda i: (0, i, 0)),
        pl.BlockSpec((1, BN, D), lambda i: (1, i, 0)),
        pl.BlockSpec((BN, 1), lambda i: (i, 0)),
        pl.BlockSpec((D, D), lambda i: (0, 0)),
        pl.BlockSpec((1, D), lambda i: (0, 0)),
        pl.BlockSpec((1, D), lambda i: (0, 0)),
        pl.BlockSpec((1, D), lambda i: (0, 0)),
    ],
    out_specs=pl.BlockSpec((BN, D), lambda i: (i, 0)),
    out_shape=jax.ShapeDtypeStruct((N, D), jnp.float32),
)


def kernel(x, edge_index, W1, b1, W2, b2, W3, b3, W4, b4, gamma, beta):
    src2 = edge_index[0].reshape(E2, CW)
    dst2 = edge_index[1].reshape(E2, CW)
    zeros_rows = jnp.zeros((ZROWS, D), jnp.float32)

    cs_flat, cd_flat = _degree_kernel(src2, dst2)
    cs = cs_flat.reshape(NC, N_PAD)[:, :N].T
    cd = cd_flat.reshape(NC, N_PAD)[:, :N].T
    h_pre, norm_src, norm_dst = _norm_scale(cs, cd, x)

    for W, b in ((W1, b1), (W2, b2), (W3, b3)):
        parts = _agg_kernel(h_pre, src2, dst2, zeros_rows)
        h_pre = _layer_mm(parts, parts, norm_dst, norm_src,
                          W, b.reshape(1, D))

    parts = _agg_kernel(h_pre, src2, dst2, zeros_rows)
    return _final_mm(parts, parts, norm_dst, W4, b4.reshape(1, D),
                     gamma.reshape(1, D), beta.reshape(1, D))


# speedup vs baseline: 1.1967x; 1.1967x over previous
import functools

import jax
import jax.numpy as jnp
from jax import lax
from jax.experimental import pallas as pl
from jax.experimental.pallas import tpu as pltpu
from jax.experimental.pallas import tpu_sc as plsc

N = 10000
E = 320000
D = 128
EPS = 1e-5

NC = 2
NS = 16
NW = NC * NS
CW = 125
E2 = E // CW
ROWS_W = E2 // NW
N_PAD = 10240
CNT_W = N_PAD // NS
AGG_W = N_PAD // NS
ZROWS = 64
ZB = 16
IDXB = 16

_mesh = plsc.VectorSubcoreMesh(core_axis_name="c", subcore_axis_name="s")


@functools.partial(
    pl.kernel,
    out_type=[
        jax.ShapeDtypeStruct((NC * N_PAD,), jnp.float32),
        jax.ShapeDtypeStruct((NC * N_PAD,), jnp.float32),
    ],
    mesh=_mesh,
    scratch_types=[
        pltpu.VMEM((ROWS_W, CW), jnp.int32),
        pltpu.VMEM((128,), jnp.float32),
        pltpu.VMEM((CNT_W,), jnp.float32),
        pltpu.VMEM_SHARED((N_PAD,), jnp.float32),
        pltpu.VMEM_SHARED((N_PAD,), jnp.float32),
    ],
)
def _degree_kernel(src_hbm, dst_hbm, out_src, out_dst,
                   idx_v, ones_v, buf_v, cs_sh, cd_sh):
    c = lax.axis_index("c")
    s = lax.axis_index("s")
    w = c * NS + s

    z16 = jnp.zeros((16,), jnp.float32)
    for i in range(CNT_W // 16):
        buf_v[pl.ds(i * 16, 16)] = z16
    o16 = jnp.ones((16,), jnp.float32)
    for i in range(128 // 16):
        ones_v[pl.ds(i * 16, 16)] = o16

    cnt0 = s * CNT_W
    pltpu.sync_copy(buf_v, cs_sh.at[pl.ds(cnt0, CNT_W)])
    pltpu.sync_copy(buf_v, cd_sh.at[pl.ds(cnt0, CNT_W)])
    plsc.subcore_barrier()

    base = w * ROWS_W
    ones_cw = ones_v.at[pl.ds(0, CW)]

    pltpu.sync_copy(src_hbm.at[pl.ds(base, ROWS_W)], idx_v)

    def body_s(j, carry):
        pltpu.sync_copy(ones_cw, cs_sh.at[idx_v.at[j]], add=True)
        return carry

    lax.fori_loop(0, ROWS_W, body_s, 0)

    pltpu.sync_copy(dst_hbm.at[pl.ds(base, ROWS_W)], idx_v)

    def body_d(j, carry):
        pltpu.sync_copy(ones_cw, cd_sh.at[idx_v.at[j]], add=True)
        return carry

    lax.fori_loop(0, ROWS_W, body_d, 0)

    plsc.subcore_barrier()
    out0 = c * N_PAD + cnt0
    pltpu.sync_copy(cs_sh.at[pl.ds(cnt0, CNT_W)], buf_v)
    pltpu.sync_copy(buf_v, out_src.at[pl.ds(out0, CNT_W)])
    pltpu.sync_copy(cd_sh.at[pl.ds(cnt0, CNT_W)], buf_v)
    pltpu.sync_copy(buf_v, out_dst.at[pl.ds(out0, CNT_W)])


@functools.partial(
    pl.kernel,
    out_type=jax.ShapeDtypeStruct((NC, N_PAD, D), jnp.float32),
    mesh=_mesh,
    scratch_types=[
        pltpu.VMEM((2, IDXB, CW), jnp.int32),
        pltpu.VMEM((2, IDXB, CW), jnp.int32),
        pltpu.VMEM((CW, D), jnp.float32),
        pltpu.VMEM((CW, D), jnp.float32),
        pltpu.VMEM((ZB, D), jnp.float32),
        pltpu.VMEM_SHARED((N_PAD, D), jnp.float32),
        pltpu.SemaphoreType.DMA,
        pltpu.SemaphoreType.DMA,
        pltpu.SemaphoreType.DMA,
        pltpu.SemaphoreType.DMA,
        pltpu.SemaphoreType.DMA,
        pltpu.SemaphoreType.DMA,
    ],
)
def _agg_kernel(h_hbm, src_hbm, dst_hbm, out_hbm,
                sidx_b, didx_b, ra, rb, zb, agg_sh, s0, s1, si, di, t0, t1):
    c = lax.axis_index("c")
    s = lax.axis_index("s")
    w = c * NS + s
    row0 = s * AGG_W
    base = w * ROWS_W
    bufs = (ra, rb)
    sems = (s0, s1)

    def idx_copy(b):
        return (
            pltpu.make_async_copy(
                src_hbm.at[pl.ds(base + b * IDXB, IDXB)], sidx_b.at[b % 2], si),
            pltpu.make_async_copy(
                dst_hbm.at[pl.ds(base + b * IDXB, IDXB)], didx_b.at[b % 2], di),
        )

    def gather_copy(j):
        b, pos = j // IDXB, j % IDXB
        return pltpu.make_async_copy(
            h_hbm.at[sidx_b.at[b % 2, pos]], bufs[j % 2], sems[j % 2])

    for cp in idx_copy(0):
        cp.start()
    for cp in idx_copy(1):
        cp.start()

    z16 = jnp.zeros((16,), jnp.float32)
    for i in range(ZB):
        for k in range(D // 16):
            zb[i, pl.ds(k * 16, 16)] = z16

    for cp in idx_copy(0):
        cp.wait()
    gather_copy(0).start()
    gather_copy(1).start()

    for i in range(AGG_W // ZB):
        pltpu.sync_copy(zb, agg_sh.at[pl.ds(row0 + i * ZB, ZB)])
    plsc.subcore_barrier()

    for j in range(ROWS_W):
        if j % IDXB == IDXB - 2 and j + 2 < ROWS_W:
            for cp in idx_copy((j + 2) // IDXB):
                cp.wait()
        if j % IDXB == 0 and 0 < j and j // IDXB + 1 < ROWS_W // IDXB:
            for cp in idx_copy(j // IDXB + 1):
                cp.start()
        gather_copy(j).wait()
        b, pos = j // IDXB, j % IDXB
        pltpu.sync_copy(bufs[j % 2], agg_sh.at[didx_b.at[b % 2, pos]], add=True)
        if j + 2 < ROWS_W:
            gather_copy(j + 2).start()

    plsc.subcore_barrier()

    def out_read(i):
        return pltpu.make_async_copy(
            agg_sh.at[pl.ds(row0 + i * ZROWS, ZROWS)],
            bufs[i % 2].at[pl.ds(0, ZROWS)], sems[i % 2])

    out_read(0).start()
    for i in range(AGG_W // ZROWS):
        if i + 1 < AGG_W // ZROWS:
            out_read(i + 1).start()
        out_read(i).wait()
        pltpu.sync_copy(bufs[i % 2].at[pl.ds(0, ZROWS)],
                        out_hbm.at[c, pl.ds(row0 + i * ZROWS, ZROWS)])


BN = 1000


def _gelu(v):
    return 0.5 * v * (1.0 + lax.erf(v * 0.7071067811865476))


def _norm_scale_body(cs_ref, cd_ref, x_ref, h_ref, ns_ref, nd_ref):
    deg_s = jnp.maximum(cs_ref[...].sum(axis=1, keepdims=True), 1.0)
    deg_d = jnp.maximum(cd_ref[...].sum(axis=1, keepdims=True), 1.0)
    ns = lax.rsqrt(deg_s)
    nd = lax.rsqrt(deg_d)
    ns_ref[...] = ns
    nd_ref[...] = nd
    h_ref[...] = x_ref[...] * ns


_norm_scale = pl.pallas_call(
    _norm_scale_body,
    grid=(N // BN,),
    in_specs=[
        pl.BlockSpec((BN, 2), lambda i: (i, 0)),
        pl.BlockSpec((BN, 2), lambda i: (i, 0)),
        pl.BlockSpec((BN, D), lambda i: (i, 0)),
    ],
    out_specs=[
        pl.BlockSpec((BN, D), lambda i: (i, 0)),
        pl.BlockSpec((BN, 1), lambda i: (i, 0)),
        pl.BlockSpec((BN, 1), lambda i: (i, 0)),
    ],
    out_shape=[
        jax.ShapeDtypeStruct((N, D), jnp.float32),
        jax.ShapeDtypeStruct((N, 1), jnp.float32),
        jax.ShapeDtypeStruct((N, 1), jnp.float32),
    ],
)


def _layer_body(p0_ref, p1_ref, nd_ref, ns_ref, w_ref, b_ref, out_ref):
    agg = (p0_ref[0] + p1_ref[0]) * nd_ref[...]
    h = jnp.dot(agg, w_ref[...], preferred_element_type=jnp.float32) + b_ref[...]
    out_ref[...] = _gelu(h) * ns_ref[...]


_layer_mm = pl.pallas_call(
    _layer_body,
    grid=(N // BN,),
    in_specs=[
        pl.BlockSpec((1, BN, D), lambda i: (0, i, 0)),
        pl.BlockSpec((1, BN, D), lambda i: (1, i, 0)),
        pl.BlockSpec((BN, 1), lambda i: (i, 0)),
        pl.BlockSpec((BN, 1), lambda i: (i, 0)),
        pl.BlockSpec((D, D), lambda i: (0, 0)),
        pl.BlockSpec((1, D), lambda i: (0, 0)),
    ],
    out_specs=pl.BlockSpec((BN, D), lambda i: (i, 0)),
    out_shape=jax.ShapeDtypeStruct((N, D), jnp.float32),
)


def _final_body(p0_ref, p1_ref, nd_ref, w_ref, b_ref, g_ref, be_ref, out_ref):
    agg = (p0_ref[0] + p1_ref[0]) * nd_ref[...]
    h = jnp.dot(agg, w_ref[...], preferred_element_type=jnp.float32) + b_ref[...]
    mu = jnp.mean(h, axis=-1, keepdims=True)
    cent = h - mu
    var = jnp.mean(cent * cent, axis=-1, keepdims=True)
    out_ref[...] = cent * lax.rsqrt(var + EPS) * g_ref[...] + be_ref[...]


_final_mm = pl.pallas_call(
    _final_body,
    grid=(N // BN,),
    in_specs=[
        pl.BlockSpec((1, BN, D), lambda i: (0, i, 0)),
        pl.BlockSpec((1, BN, D), lambda i: (1, i, 0)),
        pl.BlockSpec((BN, 1), lambda i: (i, 0)),
        pl.BlockSpec((D, D), lambda i: (0, 0)),
        pl.BlockSpec((1, D), lambda i: (0, 0)),
        pl.BlockSpec((1, D), lambda i: (0, 0)),
        pl.BlockSpec((1, D), lambda i: (0, 0)),
    ],
    out_specs=pl.BlockSpec((BN, D), lambda i: (i, 0)),
    out_shape=jax.ShapeDtypeStruct((N, D), jnp.float32),
)


def kernel(x, edge_index, W1, b1, W2, b2, W3, b3, W4, b4, gamma, beta):
    src2 = edge_index[0].reshape(E2, CW)
    dst2 = edge_index[1].reshape(E2, CW)

    cs_flat, cd_flat = _degree_kernel(src2, dst2)
    cs = cs_flat.reshape(NC, N_PAD)[:, :N].T
    cd = cd_flat.reshape(NC, N_PAD)[:, :N].T
    h_pre, norm_src, norm_dst = _norm_scale(cs, cd, x)

    for W, b in ((W1, b1), (W2, b2), (W3, b3)):
        parts = _agg_kernel(h_pre, src2, dst2)
        h_pre = _layer_mm(parts, parts, norm_dst, norm_src,
                          W, b.reshape(1, D))

    parts = _agg_kernel(h_pre, src2, dst2)
    return _final_mm(parts, parts, norm_dst, W4, b4.reshape(1, D),
                     gamma.reshape(1, D), beta.reshape(1, D))


# speedup vs baseline: 1.2231x; 1.0221x over previous
import functools

import jax
import jax.numpy as jnp
from jax import lax
from jax.experimental import pallas as pl
from jax.experimental.pallas import tpu as pltpu
from jax.experimental.pallas import tpu_sc as plsc

N = 10000
E = 320000
D = 128
EPS = 1e-5

NC = 2
NS = 16
NW = NC * NS
CW = 125
E2 = E // CW
ROWS_W = E2 // NW
N_PAD = 10240
CNT_W = N_PAD // NS
AGG_W = N_PAD // NS
ZROWS = 64
ZB = 16
IDXB = 16

_mesh = plsc.VectorSubcoreMesh(core_axis_name="c", subcore_axis_name="s")


@functools.partial(
    pl.kernel,
    out_type=[
        jax.ShapeDtypeStruct((NC * N_PAD,), jnp.float32),
        jax.ShapeDtypeStruct((NC * N_PAD,), jnp.float32),
    ],
    mesh=_mesh,
    scratch_types=[
        pltpu.VMEM((ROWS_W, CW), jnp.int32),
        pltpu.VMEM((ROWS_W, CW), jnp.int32),
        pltpu.VMEM((128,), jnp.float32),
        pltpu.VMEM((CNT_W,), jnp.float32),
        pltpu.VMEM_SHARED((N_PAD,), jnp.float32),
        pltpu.VMEM_SHARED((N_PAD,), jnp.float32),
        pltpu.SemaphoreType.DMA,
        pltpu.SemaphoreType.DMA,
        pltpu.SemaphoreType.DMA,
        pltpu.SemaphoreType.DMA,
    ],
)
def _degree_kernel(src_hbm, dst_hbm, out_src, out_dst,
                   sidx_v, didx_v, ones_v, buf_v, cs_sh, cd_sh,
                   si, di, ss, sd):
    c = lax.axis_index("c")
    s = lax.axis_index("s")
    w = c * NS + s
    base = w * ROWS_W

    ci = pltpu.make_async_copy(src_hbm.at[pl.ds(base, ROWS_W)], sidx_v, si)
    cj = pltpu.make_async_copy(dst_hbm.at[pl.ds(base, ROWS_W)], didx_v, di)
    ci.start()
    cj.start()

    z16 = jnp.zeros((16,), jnp.float32)
    for i in range(CNT_W // 16):
        buf_v[pl.ds(i * 16, 16)] = z16
    o16 = jnp.ones((16,), jnp.float32)
    for i in range(128 // 16):
        ones_v[pl.ds(i * 16, 16)] = o16

    cnt0 = s * CNT_W
    pltpu.sync_copy(buf_v, cs_sh.at[pl.ds(cnt0, CNT_W)])
    pltpu.sync_copy(buf_v, cd_sh.at[pl.ds(cnt0, CNT_W)])
    ci.wait()
    cj.wait()
    plsc.subcore_barrier()

    ones_cw = ones_v.at[pl.ds(0, CW)]

    def body_fire(j, carry):
        pltpu.make_async_copy(ones_cw, cs_sh.at[sidx_v.at[j]], ss).start(add=True)
        pltpu.make_async_copy(ones_cw, cd_sh.at[didx_v.at[j]], sd).start(add=True)
        return carry

    lax.fori_loop(0, ROWS_W, body_fire, 0)

    def body_drain(j, carry):
        pltpu.make_async_copy(ones_cw, cs_sh.at[sidx_v.at[j]], ss).wait()
        pltpu.make_async_copy(ones_cw, cd_sh.at[didx_v.at[j]], sd).wait()
        return carry

    lax.fori_loop(0, ROWS_W, body_drain, 0)

    plsc.subcore_barrier()
    out0 = c * N_PAD + cnt0
    pltpu.sync_copy(cs_sh.at[pl.ds(cnt0, CNT_W)], buf_v)
    pltpu.sync_copy(buf_v, out_src.at[pl.ds(out0, CNT_W)])
    pltpu.sync_copy(cd_sh.at[pl.ds(cnt0, CNT_W)], buf_v)
    pltpu.sync_copy(buf_v, out_dst.at[pl.ds(out0, CNT_W)])


@functools.partial(
    pl.kernel,
    out_type=jax.ShapeDtypeStruct((NC, N_PAD, D), jnp.float32),
    mesh=_mesh,
    scratch_types=[
        pltpu.VMEM((2, IDXB, CW), jnp.int32),
        pltpu.VMEM((2, IDXB, CW), jnp.int32),
        pltpu.VMEM((CW, D), jnp.float32),
        pltpu.VMEM((CW, D), jnp.float32),
        pltpu.VMEM((ZB, D), jnp.float32),
        pltpu.VMEM_SHARED((N_PAD, D), jnp.float32),
        pltpu.SemaphoreType.DMA,
        pltpu.SemaphoreType.DMA,
        pltpu.SemaphoreType.DMA,
        pltpu.SemaphoreType.DMA,
        pltpu.SemaphoreType.DMA,
        pltpu.SemaphoreType.DMA,
    ],
)
def _agg_kernel(h_hbm, src_hbm, dst_hbm, out_hbm,
                sidx_b, didx_b, ra, rb, zb, agg_sh, s0, s1, si, di, t0, t1):
    c = lax.axis_index("c")
    s = lax.axis_index("s")
    w = c * NS + s
    row0 = s * AGG_W
    base = w * ROWS_W
    bufs = (ra, rb)
    sems = (s0, s1)

    def idx_copy(b):
        return (
            pltpu.make_async_copy(
                src_hbm.at[pl.ds(base + b * IDXB, IDXB)], sidx_b.at[b % 2], si),
            pltpu.make_async_copy(
                dst_hbm.at[pl.ds(base + b * IDXB, IDXB)], didx_b.at[b % 2], di),
        )

    def gather_copy(j):
        b, pos = j // IDXB, j % IDXB
        return pltpu.make_async_copy(
            h_hbm.at[sidx_b.at[b % 2, pos]], bufs[j % 2], sems[j % 2])

    for cp in idx_copy(0):
        cp.start()
    for cp in idx_copy(1):
        cp.start()

    z16 = jnp.zeros((16,), jnp.float32)
    for i in range(ZB):
        for k in range(D // 16):
            zb[i, pl.ds(k * 16, 16)] = z16

    for cp in idx_copy(0):
        cp.wait()
    gather_copy(0).start()
    gather_copy(1).start()

    for i in range(AGG_W // ZB):
        pltpu.sync_copy(zb, agg_sh.at[pl.ds(row0 + i * ZB, ZB)])
    plsc.subcore_barrier()

    for j in range(ROWS_W):
        if j % IDXB == IDXB - 2 and j + 2 < ROWS_W:
            for cp in idx_copy((j + 2) // IDXB):
                cp.wait()
        if j % IDXB == 0 and 0 < j and j // IDXB + 1 < ROWS_W // IDXB:
            for cp in idx_copy(j // IDXB + 1):
                cp.start()
        gather_copy(j).wait()
        b, pos = j // IDXB, j % IDXB
        pltpu.sync_copy(bufs[j % 2], agg_sh.at[didx_b.at[b % 2, pos]], add=True)
        if j + 2 < ROWS_W:
            gather_copy(j + 2).start()

    plsc.subcore_barrier()

    def out_read(i):
        return pltpu.make_async_copy(
            agg_sh.at[pl.ds(row0 + i * ZROWS, ZROWS)],
            bufs[i % 2].at[pl.ds(0, ZROWS)], sems[i % 2])

    out_read(0).start()
    for i in range(AGG_W // ZROWS):
        if i + 1 < AGG_W // ZROWS:
            out_read(i + 1).start()
        out_read(i).wait()
        pltpu.sync_copy(bufs[i % 2].at[pl.ds(0, ZROWS)],
                        out_hbm.at[c, pl.ds(row0 + i * ZROWS, ZROWS)])


BN = 1000


def _gelu(v):
    return 0.5 * v * (1.0 + lax.erf(v * 0.7071067811865476))


def _norm_scale_body(cs_ref, cd_ref, x_ref, h_ref, ns_ref, nd_ref):
    deg_s = jnp.maximum(cs_ref[...].sum(axis=1, keepdims=True), 1.0)
    deg_d = jnp.maximum(cd_ref[...].sum(axis=1, keepdims=True), 1.0)
    ns = lax.rsqrt(deg_s)
    nd = lax.rsqrt(deg_d)
    ns_ref[...] = ns
    nd_ref[...] = nd
    h_ref[...] = x_ref[...] * ns


_norm_scale = pl.pallas_call(
    _norm_scale_body,
    grid=(N // BN,),
    in_specs=[
        pl.BlockSpec((BN, 2), lambda i: (i, 0)),
        pl.BlockSpec((BN, 2), lambda i: (i, 0)),
        pl.BlockSpec((BN, D), lambda i: (i, 0)),
    ],
    out_specs=[
        pl.BlockSpec((BN, D), lambda i: (i, 0)),
        pl.BlockSpec((BN, 1), lambda i: (i, 0)),
        pl.BlockSpec((BN, 1), lambda i: (i, 0)),
    ],
    out_shape=[
        jax.ShapeDtypeStruct((N, D), jnp.float32),
        jax.ShapeDtypeStruct((N, 1), jnp.float32),
        jax.ShapeDtypeStruct((N, 1), jnp.float32),
    ],
)


def _layer_body(p0_ref, p1_ref, nd_ref, ns_ref, w_ref, b_ref, out_ref):
    agg = (p0_ref[0] + p1_ref[0]) * nd_ref[...]
    h = jnp.dot(agg, w_ref[...], preferred_element_type=jnp.float32) + b_ref[...]
    out_ref[...] = _gelu(h) * ns_ref[...]


_layer_mm = pl.pallas_call(
    _layer_body,
    grid=(N // BN,),
    in_specs=[
        pl.BlockSpec((1, BN, D), lambda i: (0, i, 0)),
        pl.BlockSpec((1, BN, D), lambda i: (1, i, 0)),
        pl.BlockSpec((BN, 1), lambda i: (i, 0)),
        pl.BlockSpec((BN, 1), lambda i: (i, 0)),
        pl.BlockSpec((D, D), lambda i: (0, 0)),
        pl.BlockSpec((1, D), lambda i: (0, 0)),
    ],
    out_specs=pl.BlockSpec((BN, D), lambda i: (i, 0)),
    out_shape=jax.ShapeDtypeStruct((N, D), jnp.float32),
)


def _final_body(p0_ref, p1_ref, nd_ref, w_ref, b_ref, g_ref, be_ref, out_ref):
    agg = (p0_ref[0] + p1_ref[0]) * nd_ref[...]
    h = jnp.dot(agg, w_ref[...], preferred_element_type=jnp.float32) + b_ref[...]
    mu = jnp.mean(h, axis=-1, keepdims=True)
    cent = h - mu
    var = jnp.mean(cent * cent, axis=-1, keepdims=True)
    out_ref[...] = cent * lax.rsqrt(var + EPS) * g_ref[...] + be_ref[...]


_final_mm = pl.pallas_call(
    _final_body,
    grid=(N // BN,),
    in_specs=[
        pl.BlockSpec((1, BN, D), lambda i: (0, i, 0)),
        pl.BlockSpec((1, BN, D), lambda i: (1, i, 0)),
        pl.BlockSpec((BN, 1), lambda i: (i, 0)),
        pl.BlockSpec((D, D), lambda i: (0, 0)),
        pl.BlockSpec((1, D), lambda i: (0, 0)),
        pl.BlockSpec((1, D), lambda i: (0, 0)),
        pl.BlockSpec((1, D), lambda i: (0, 0)),
    ],
    out_specs=pl.BlockSpec((BN, D), lambda i: (i, 0)),
    out_shape=jax.ShapeDtypeStruct((N, D), jnp.float32),
)


def kernel(x, edge_index, W1, b1, W2, b2, W3, b3, W4, b4, gamma, beta):
    src2 = edge_index[0].reshape(E2, CW)
    dst2 = edge_index[1].reshape(E2, CW)

    cs_flat, cd_flat = _degree_kernel(src2, dst2)
    cs = cs_flat.reshape(NC, N_PAD)[:, :N].T
    cd = cd_flat.reshape(NC, N_PAD)[:, :N].T
    h_pre, norm_src, norm_dst = _norm_scale(cs, cd, x)

    for W, b in ((W1, b1), (W2, b2), (W3, b3)):
        parts = _agg_kernel(h_pre, src2, dst2)
        h_pre = _layer_mm(parts, parts, norm_dst, norm_src,
                          W, b.reshape(1, D))

    parts = _agg_kernel(h_pre, src2, dst2)
    return _final_mm(parts, parts, norm_dst, W4, b4.reshape(1, D),
                     gamma.reshape(1, D), beta.reshape(1, D))


# speedup vs baseline: 1.2386x; 1.0126x over previous
import functools

import jax
import jax.numpy as jnp
from jax import lax
from jax.experimental import pallas as pl
from jax.experimental.pallas import tpu as pltpu
from jax.experimental.pallas import tpu_sc as plsc

N = 10000
E = 320000
D = 128
EPS = 1e-5

NC = 2
NS = 16
NW = NC * NS
CW = 125
E2 = E // CW
ROWS_W = E2 // NW
N_PAD = 10240
CNT_W = N_PAD // NS
AGG_W = N_PAD // NS
ZROWS = 64
ZB = 16
IDXB = 16

_mesh = plsc.VectorSubcoreMesh(core_axis_name="c", subcore_axis_name="s")


@functools.partial(
    pl.kernel,
    out_type=[
        jax.ShapeDtypeStruct((NC * N_PAD,), jnp.float32),
        jax.ShapeDtypeStruct((NC * N_PAD,), jnp.float32),
    ],
    mesh=_mesh,
    scratch_types=[
        pltpu.VMEM((ROWS_W, CW), jnp.int32),
        pltpu.VMEM((ROWS_W, CW), jnp.int32),
        pltpu.VMEM((128,), jnp.float32),
        pltpu.VMEM((CNT_W,), jnp.float32),
        pltpu.VMEM_SHARED((N_PAD,), jnp.float32),
        pltpu.VMEM_SHARED((N_PAD,), jnp.float32),
        pltpu.SemaphoreType.DMA,
        pltpu.SemaphoreType.DMA,
        pltpu.SemaphoreType.DMA,
        pltpu.SemaphoreType.DMA,
    ],
)
def _degree_kernel(src_hbm, dst_hbm, out_src, out_dst,
                   sidx_v, didx_v, ones_v, buf_v, cs_sh, cd_sh,
                   si, di, ss, sd):
    c = lax.axis_index("c")
    s = lax.axis_index("s")
    w = c * NS + s
    base = w * ROWS_W

    ci = pltpu.make_async_copy(src_hbm.at[pl.ds(base, ROWS_W)], sidx_v, si)
    cj = pltpu.make_async_copy(dst_hbm.at[pl.ds(base, ROWS_W)], didx_v, di)
    ci.start()
    cj.start()

    z16 = jnp.zeros((16,), jnp.float32)
    for i in range(CNT_W // 16):
        buf_v[pl.ds(i * 16, 16)] = z16
    o16 = jnp.ones((16,), jnp.float32)
    for i in range(128 // 16):
        ones_v[pl.ds(i * 16, 16)] = o16

    cnt0 = s * CNT_W
    pltpu.sync_copy(buf_v, cs_sh.at[pl.ds(cnt0, CNT_W)])
    pltpu.sync_copy(buf_v, cd_sh.at[pl.ds(cnt0, CNT_W)])
    ci.wait()
    cj.wait()
    plsc.subcore_barrier()

    ones_cw = ones_v.at[pl.ds(0, CW)]

    def body_fire(j, carry):
        pltpu.make_async_copy(ones_cw, cs_sh.at[sidx_v.at[j]], ss).start(add=True)
        pltpu.make_async_copy(ones_cw, cd_sh.at[didx_v.at[j]], sd).start(add=True)
        return carry

    lax.fori_loop(0, ROWS_W, body_fire, 0)

    def body_drain(j, carry):
        pltpu.make_async_copy(ones_cw, cs_sh.at[sidx_v.at[j]], ss).wait()
        pltpu.make_async_copy(ones_cw, cd_sh.at[didx_v.at[j]], sd).wait()
        return carry

    lax.fori_loop(0, ROWS_W, body_drain, 0)

    plsc.subcore_barrier()
    out0 = c * N_PAD + cnt0
    pltpu.sync_copy(cs_sh.at[pl.ds(cnt0, CNT_W)], buf_v)
    pltpu.sync_copy(buf_v, out_src.at[pl.ds(out0, CNT_W)])
    pltpu.sync_copy(cd_sh.at[pl.ds(cnt0, CNT_W)], buf_v)
    pltpu.sync_copy(buf_v, out_dst.at[pl.ds(out0, CNT_W)])


@functools.partial(
    pl.kernel,
    out_type=jax.ShapeDtypeStruct((NC, N_PAD, D), jnp.float32),
    mesh=_mesh,
    scratch_types=[
        pltpu.VMEM((2, IDXB, CW), jnp.int32),
        pltpu.VMEM((2, IDXB, CW), jnp.int32),
        pltpu.VMEM((CW, D), jnp.float32),
        pltpu.VMEM((CW, D), jnp.float32),
        pltpu.VMEM((ZB, D), jnp.float32),
        pltpu.VMEM_SHARED((N_PAD, D), jnp.float32),
        pltpu.SemaphoreType.DMA,
        pltpu.SemaphoreType.DMA,
        pltpu.SemaphoreType.DMA,
        pltpu.SemaphoreType.DMA,
        pltpu.SemaphoreType.DMA,
        pltpu.SemaphoreType.DMA,
    ],
)
def _agg_kernel(h_hbm, src_hbm, dst_hbm, out_hbm,
                sidx_b, didx_b, ra, rb, zb, agg_sh, s0, s1, si, di, t0, t1):
    c = lax.axis_index("c")
    s = lax.axis_index("s")
    w = c * NS + s
    row0 = s * AGG_W
    base = w * ROWS_W
    bufs = (ra, rb)
    sems = (s0, s1)

    def idx_copy(b):
        return (
            pltpu.make_async_copy(
                src_hbm.at[pl.ds(base + b * IDXB, IDXB)], sidx_b.at[b % 2], si),
            pltpu.make_async_copy(
                dst_hbm.at[pl.ds(base + b * IDXB, IDXB)], didx_b.at[b % 2], di),
        )

    def gather_copy(j):
        b, pos = j // IDXB, j % IDXB
        return pltpu.make_async_copy(
            h_hbm.at[sidx_b.at[b % 2, pos]], bufs[j % 2], sems[j % 2])

    for cp in idx_copy(0):
        cp.start()
    for cp in idx_copy(1):
        cp.start()

    z16 = jnp.zeros((16,), jnp.float32)
    for i in range(ZB):
        for k in range(D // 16):
            zb[i, pl.ds(k * 16, 16)] = z16

    for cp in idx_copy(0):
        cp.wait()
    gather_copy(0).start()
    gather_copy(1).start()

    def zero_copy(i):
        return pltpu.make_async_copy(
            zb, agg_sh.at[pl.ds(row0 + i * ZB, ZB)], t0)

    for i in range(AGG_W // ZB):
        zero_copy(i).start()
    for i in range(AGG_W // ZB):
        zero_copy(i).wait()
    plsc.subcore_barrier()

    for j in range(ROWS_W):
        if j % IDXB == IDXB - 2 and j + 2 < ROWS_W:
            for cp in idx_copy((j + 2) // IDXB):
                cp.wait()
        if j % IDXB == 0 and 0 < j and j // IDXB + 1 < ROWS_W // IDXB:
            for cp in idx_copy(j // IDXB + 1):
                cp.start()
        gather_copy(j).wait()
        b, pos = j // IDXB, j % IDXB
        pltpu.sync_copy(bufs[j % 2], agg_sh.at[didx_b.at[b % 2, pos]], add=True)
        if j + 2 < ROWS_W:
            gather_copy(j + 2).start()

    plsc.subcore_barrier()

    def out_read(i):
        return pltpu.make_async_copy(
            agg_sh.at[pl.ds(row0 + i * ZROWS, ZROWS)],
            bufs[i % 2].at[pl.ds(0, ZROWS)], sems[i % 2])

    out_read(0).start()
    for i in range(AGG_W // ZROWS):
        if i + 1 < AGG_W // ZROWS:
            out_read(i + 1).start()
        out_read(i).wait()
        pltpu.sync_copy(bufs[i % 2].at[pl.ds(0, ZROWS)],
                        out_hbm.at[c, pl.ds(row0 + i * ZROWS, ZROWS)])


BN = 1000


def _gelu(v):
    return 0.5 * v * (1.0 + lax.erf(v * 0.7071067811865476))


def _norm_scale_body(cs_ref, cd_ref, x_ref, h_ref, ns_ref, nd_ref):
    deg_s = jnp.maximum(cs_ref[...].sum(axis=1, keepdims=True), 1.0)
    deg_d = jnp.maximum(cd_ref[...].sum(axis=1, keepdims=True), 1.0)
    ns = lax.rsqrt(deg_s)
    nd = lax.rsqrt(deg_d)
    ns_ref[...] = ns
    nd_ref[...] = nd
    h_ref[...] = x_ref[...] * ns


_norm_scale = pl.pallas_call(
    _norm_scale_body,
    grid=(N // BN,),
    in_specs=[
        pl.BlockSpec((BN, 2), lambda i: (i, 0)),
        pl.BlockSpec((BN, 2), lambda i: (i, 0)),
        pl.BlockSpec((BN, D), lambda i: (i, 0)),
    ],
    out_specs=[
        pl.BlockSpec((BN, D), lambda i: (i, 0)),
        pl.BlockSpec((BN, 1), lambda i: (i, 0)),
        pl.BlockSpec((BN, 1), lambda i: (i, 0)),
    ],
    out_shape=[
        jax.ShapeDtypeStruct((N, D), jnp.float32),
        jax.ShapeDtypeStruct((N, 1), jnp.float32),
        jax.ShapeDtypeStruct((N, 1), jnp.float32),
    ],
)


def _layer_body(p0_ref, p1_ref, nd_ref, ns_ref, w_ref, b_ref, out_ref):
    agg = (p0_ref[0] + p1_ref[0]) * nd_ref[...]
    h = jnp.dot(agg, w_ref[...], preferred_element_type=jnp.float32) + b_ref[...]
    out_ref[...] = _gelu(h) * ns_ref[...]


_layer_mm = pl.pallas_call(
    _layer_body,
    grid=(N // BN,),
    in_specs=[
        pl.BlockSpec((1, BN, D), lambda i: (0, i, 0)),
        pl.BlockSpec((1, BN, D), lambda i: (1, i, 0)),
        pl.BlockSpec((BN, 1), lambda i: (i, 0)),
        pl.BlockSpec((BN, 1), lambda i: (i, 0)),
        pl.BlockSpec((D, D), lambda i: (0, 0)),
        pl.BlockSpec((1, D), lambda i: (0, 0)),
    ],
    out_specs=pl.BlockSpec((BN, D), lambda i: (i, 0)),
    out_shape=jax.ShapeDtypeStruct((N, D), jnp.float32),
)


def _final_body(p0_ref, p1_ref, nd_ref, w_ref, b_ref, g_ref, be_ref, out_ref):
    agg = (p0_ref[0] + p1_ref[0]) * nd_ref[...]
    h = jnp.dot(agg, w_ref[...], preferred_element_type=jnp.float32) + b_ref[...]
    mu = jnp.mean(h, axis=-1, keepdims=True)
    cent = h - mu
    var = jnp.mean(cent * cent, axis=-1, keepdims=True)
    out_ref[...] = cent * lax.rsqrt(var + EPS) * g_ref[...] + be_ref[...]


_final_mm = pl.pallas_call(
    _final_body,
    grid=(N // BN,),
    in_specs=[
        pl.BlockSpec((1, BN, D), lambda i: (0, i, 0)),
        pl.BlockSpec((1, BN, D), lambda i: (1, i, 0)),
        pl.BlockSpec((BN, 1), lambda i: (i, 0)),
        pl.BlockSpec((D, D), lambda i: (0, 0)),
        pl.BlockSpec((1, D), lambda i: (0, 0)),
        pl.BlockSpec((1, D), lambda i: (0, 0)),
        pl.BlockSpec((1, D), lambda i: (0, 0)),
    ],
    out_specs=pl.BlockSpec((BN, D), lambda i: (i, 0)),
    out_shape=jax.ShapeDtypeStruct((N, D), jnp.float32),
)


def kernel(x, edge_index, W1, b1, W2, b2, W3, b3, W4, b4, gamma, beta):
    src2 = edge_index[0].reshape(E2, CW)
    dst2 = edge_index[1].reshape(E2, CW)

    cs_flat, cd_flat = _degree_kernel(src2, dst2)
    cs = cs_flat.reshape(NC, N_PAD)[:, :N].T
    cd = cd_flat.reshape(NC, N_PAD)[:, :N].T
    h_pre, norm_src, norm_dst = _norm_scale(cs, cd, x)

    for W, b in ((W1, b1), (W2, b2), (W3, b3)):
        parts = _agg_kernel(h_pre, src2, dst2)
        h_pre = _layer_mm(parts, parts, norm_dst, norm_src,
                          W, b.reshape(1, D))

    parts = _agg_kernel(h_pre, src2, dst2)
    return _final_mm(parts, parts, norm_dst, W4, b4.reshape(1, D),
                     gamma.reshape(1, D), beta.reshape(1, D))
